# baseline (speedup 1.0000x reference)
import jax
import jax.numpy as jnp
from jax.experimental import pallas as pl

B, Q, T, C = 128, 900, 256, 25
NUM_SELECT = 300
CPAD = 128


def _prob_kernel(logits_ref, pmT_ref, prob_ref, rmax_ref):
    s = jax.nn.sigmoid(logits_ref[0])
    p = jnp.dot(s, pmT_ref[...], preferred_element_type=jnp.float32)
    prob_ref[0] = p[:, :C]
    rmax_ref[0, 0] = jnp.max(p, axis=1)


def kernel(pred_logits, pred_boxes, target_sizes, positive_map):
    pmT = jnp.zeros((T, CPAD), jnp.float32).at[:, :C].set(positive_map.T)
    prob, rowmax = pl.pallas_call(
        _prob_kernel,
        grid=(B,),
        in_specs=[
            pl.BlockSpec((1, Q, T), lambda b: (b, 0, 0)),
            pl.BlockSpec((T, CPAD), lambda b: (0, 0)),
        ],
        out_specs=[
            pl.BlockSpec((1, Q, C), lambda b: (b, 0, 0)),
            pl.BlockSpec((1, 1, Q), lambda b: (b, 0, 0)),
        ],
        out_shape=[
            jax.ShapeDtypeStruct((B, Q, C), jnp.float32),
            jax.ShapeDtypeStruct((B, 1, Q), jnp.float32),
        ],
    )(pred_logits, pmT)

    _, q_cand = jax.lax.top_k(rowmax[:, 0, :], NUM_SELECT)
    q_cand = jnp.sort(q_cand, axis=1)

    W = 7552
    cand = jnp.take_along_axis(
        prob, q_cand[:, :, None], axis=1)
    flat = cand.reshape(B, NUM_SELECT * C)
    neg = jnp.full((B, W), 1.0, jnp.float32).at[:, :NUM_SELECT * C].set(-flat)
    iota = jnp.broadcast_to(jax.lax.iota(jnp.int32, W), (B, W))
    sneg, spos = jax.lax.sort((neg, iota), dimension=1, num_keys=1,
                              is_stable=True)
    scores = -sneg[:, :NUM_SELECT]
    pos = spos[:, :NUM_SELECT]
    topk_boxes = jnp.take_along_axis(q_cand, pos // C, axis=1)
    labels = pos % C

    cx, cy, w, h = (pred_boxes[..., i] for i in range(4))
    boxes = jnp.stack([cx - 0.5 * w, cy - 0.5 * h, cx + 0.5 * w, cy + 0.5 * h], -1)
    idx = jnp.broadcast_to(topk_boxes[:, :, None], (B, NUM_SELECT, 4))
    boxes = jnp.take_along_axis(boxes, idx, axis=1)
    img_h = target_sizes[:, 0].astype(boxes.dtype)
    img_w = target_sizes[:, 1].astype(boxes.dtype)
    scale_fct = jnp.stack([img_w, img_h, img_w, img_h], axis=1)
    boxes = boxes * scale_fct[:, None, :]
    return scores, labels, boxes
